# Initial kernel scaffold; baseline (speedup 1.0000x reference)
import jax
import jax.numpy as jnp
from jax.experimental import pallas as pl


def kernel(vectors, node_specie, senders, receivers, embed_table, W_lin, Wr1, Wr2, W_sh, W_prod_sp, W_skip_sp, W_mix, W_readout):
    raise NotImplementedError("write your pallas kernel here")


# speedup vs baseline: 1.6900x; 1.6900x over previous
import functools

import jax
import jax.numpy as jnp
from jax import lax
from jax.experimental import pallas as pl
from jax.experimental.pallas import tpu as pltpu
from jax.experimental.pallas import tpu_sc as plsc

N = 10000
E = 320000
F = 128
H = 64
NUM_SPECIES = 10
NUM_BESSEL = 8
R_MAX = 5.0
NUM_LAYERS = 2
AVG_NEIGH = 32.0
H_RAD = 64

NCORE = 2
NSUB = 16
CH = 125
ROWS = E // CH
CPS = ROWS // NSUB
IB = 40
NB = CPS // IB
RPS = N // NSUB

BE = 3200
BN = 2000
F32 = jnp.float32


def _edge_factor_body(vec_ref, wr1a_ref, wr2a_ref, wsha_ref,
                      wr1b_ref, wr2b_ref, wshb_ref, rg0_ref, rg1_ref):
    v = vec_ref[...]
    r2 = jnp.sum(v * v, axis=1, keepdims=True)
    r = jnp.sqrt(r2)
    rc = jnp.maximum(r, 1e-6)
    unit = v / rc
    sh = jnp.concatenate(
        [jnp.full((v.shape[0], 1), 0.28209479177387814, F32),
         0.4886025119029199 * unit], axis=1)
    n = (lax.broadcasted_iota(jnp.int32, (1, NUM_BESSEL), 1) + 1).astype(F32)
    b = jnp.sqrt(2.0 / R_MAX) * jnp.sin(n * (jnp.pi / R_MAX) * rc) / rc
    u = r / R_MAX
    u5 = u * u * u * u * u
    env = 1.0 - 21.0 * u5 + 35.0 * u5 * u - 15.0 * u5 * u * u
    env = jnp.where(u < 1.0, env, 0.0)
    rad = b * env

    for wr1_ref, wr2_ref, wsh_ref, out_ref in (
            (wr1a_ref, wr2a_ref, wsha_ref, rg0_ref),
            (wr1b_ref, wr2b_ref, wshb_ref, rg1_ref)):
        z = jnp.dot(rad, wr1_ref[...], preferred_element_type=F32)
        z = z * jax.nn.sigmoid(z)
        rw = jnp.dot(z, wr2_ref[...], preferred_element_type=F32)
        gate = jnp.dot(sh, wsh_ref[...], preferred_element_type=F32)
        rg = rw * gate
        out_ref[0, :, :] = rg[:, :H]
        out_ref[1, :, :] = rg[:, H:]


def _edge_factors(vectors, Wr1, Wr2, W_sh):
    w_spec = lambda s: pl.BlockSpec(s, lambda i: (0,) * len(s))
    grid = E // BE
    return pl.pallas_call(
        _edge_factor_body,
        grid=(grid,),
        in_specs=[
            pl.BlockSpec((BE, 3), lambda i: (i, 0)),
            w_spec((NUM_BESSEL, H_RAD)), w_spec((H_RAD, F)), w_spec((4, F)),
            w_spec((NUM_BESSEL, H_RAD)), w_spec((H_RAD, F)), w_spec((4, F)),
        ],
        out_specs=[
            pl.BlockSpec((2, BE, H), lambda i: (0, i, 0)),
            pl.BlockSpec((2, BE, H), lambda i: (0, i, 0)),
        ],
        out_shape=[jax.ShapeDtypeStruct((2, E, H), F32)] * 2,
    )(vectors, Wr1[0], Wr2[0], W_sh[0], Wr1[1], Wr2[1], W_sh[1])


def _embed_body(spec_ref, emb_ref, embw_ref, nf_ref, h_ref):
    s = spec_ref[0, 0, :]
    oh = (s[:, None] == lax.broadcasted_iota(
        jnp.int32, (s.shape[0], NUM_SPECIES), 1)).astype(F32)
    nf = jnp.dot(oh, emb_ref[...], preferred_element_type=F32)
    h = jnp.dot(oh, embw_ref[...], preferred_element_type=F32)
    nf_ref[...] = nf
    h_ref[0, :, :] = h[:, :H]
    h_ref[1, :, :] = h[:, H:]


def _embed(spec3d, embed_table, embw0):
    grid = N // BN
    return pl.pallas_call(
        _embed_body,
        grid=(grid,),
        in_specs=[
            pl.BlockSpec((1, 1, BN), lambda i: (i, 0, 0)),
            pl.BlockSpec((NUM_SPECIES, F), lambda i: (0, 0)),
            pl.BlockSpec((NUM_SPECIES, F), lambda i: (0, 0)),
        ],
        out_specs=[
            pl.BlockSpec((BN, F), lambda i: (i, 0)),
            pl.BlockSpec((2, BN, H), lambda i: (0, i, 0)),
        ],
        out_shape=[jax.ShapeDtypeStruct((N, F), F32),
                   jax.ShapeDtypeStruct((2, N, H), F32)],
    )(spec3d, embed_table, embw0)


def _sc_body(h_hbm, rg_hbm, snd_hbm, rcv_hbm, agg_hbm,
             h_sh, agg_sh, snd_v, rcv_v, rows_v, rg_v):
    cid = lax.axis_index("c")
    sid = lax.axis_index("s")

    pltpu.sync_copy(h_hbm.at[cid, pl.ds(sid * RPS, RPS)],
                    h_sh.at[pl.ds(sid * RPS, RPS)])

    def _zero_row(r, carry):
        for k in range(H // 16):
            rows_v[r, pl.ds(k * 16, 16)] = jnp.zeros((16,), F32)
        return carry
    lax.fori_loop(0, CH, _zero_row, 0)
    for t in range(RPS // CH):
        pltpu.sync_copy(rows_v, agg_sh.at[pl.ds(sid * RPS + t * CH, CH)])

    plsc.subcore_barrier()

    def _batch(b, carry):
        pltpu.sync_copy(snd_hbm.at[pl.ds(sid * CPS + b * IB, IB)], snd_v)
        pltpu.sync_copy(rcv_hbm.at[pl.ds(sid * CPS + b * IB, IB)], rcv_v)

        def _edge_chunk(j, c1):
            pltpu.sync_copy(h_sh.at[snd_v.at[j]], rows_v)
            pltpu.sync_copy(
                rg_hbm.at[cid, pl.ds((sid * CPS + b * IB + j) * CH, CH)],
                rg_v)

            def _mul_row(r, c2):
                for k in range(H // 16):
                    sl = pl.ds(k * 16, 16)
                    rows_v[r, sl] = rows_v[r, sl] * rg_v[r, sl]
                return c2
            lax.fori_loop(0, CH, _mul_row, 0)

            pltpu.sync_copy(rows_v, agg_sh.at[rcv_v.at[j]], add=True)
            return c1
        lax.fori_loop(0, IB, _edge_chunk, 0)
        return carry
    lax.fori_loop(0, NB, _batch, 0)

    plsc.subcore_barrier()
    pltpu.sync_copy(agg_sh.at[pl.ds(sid * RPS, RPS)],
                    agg_hbm.at[cid, pl.ds(sid * RPS, RPS)])


def _sc_message(h2, rg, snd2d, rcv2d):
    mesh = plsc.VectorSubcoreMesh(core_axis_name="c", subcore_axis_name="s",
                                  num_cores=NCORE, num_subcores=NSUB)
    return pl.kernel(
        _sc_body,
        out_type=jax.ShapeDtypeStruct((2, N, H), F32),
        mesh=mesh,
        compiler_params=pltpu.CompilerParams(use_tc_tiling_on_sc=False),
        scratch_types=[
            pltpu.VMEM_SHARED((N, H), F32),
            pltpu.VMEM_SHARED((N, H), F32),
            pltpu.VMEM((IB, CH), jnp.int32),
            pltpu.VMEM((IB, CH), jnp.int32),
            pltpu.VMEM((CH, H), F32),
            pltpu.VMEM((CH, H), F32),
        ],
    )(h2, rg, snd2d, rcv2d)


def _node_body(has_next, agg_ref, nf_ref, spec_ref, wp_ref, ws_ref,
               wmix_ref, wread_ref, *rest):
    if has_next:
        wlin_ref, nfo_ref, out_ref, hn_ref = rest
    else:
        nfo_ref, out_ref = rest
    a = jnp.concatenate([agg_ref[0], agg_ref[1]], axis=1) * (1.0 / AVG_NEIGH)
    poly = a + a * a + a * a * a
    s = spec_ref[0, 0, :]
    oh = (s[:, None] == lax.broadcasted_iota(
        jnp.int32, (s.shape[0], NUM_SPECIES), 1)).astype(F32)
    wp = jnp.dot(oh, wp_ref[...], preferred_element_type=F32)
    ws = jnp.dot(oh, ws_ref[...], preferred_element_type=F32)
    node_new = jnp.dot(poly * wp, wmix_ref[...], preferred_element_type=F32)
    nf = node_new + nf_ref[...] * ws
    nfo_ref[...] = nf
    out_ref[...] = jnp.dot(nf, wread_ref[...], preferred_element_type=F32)
    if has_next:
        hn = jnp.dot(nf, wlin_ref[...], preferred_element_type=F32)
        hn_ref[0, :, :] = hn[:, :H]
        hn_ref[1, :, :] = hn[:, H:]


def _node_update(agg, nf, spec3d, wp, ws, wmix, wread, wlin_next):
    has_next = wlin_next is not None
    grid = N // BN
    w_spec = lambda s: pl.BlockSpec(s, lambda i: (0,) * len(s))
    in_specs = [
        pl.BlockSpec((2, BN, H), lambda i: (0, i, 0)),
        pl.BlockSpec((BN, F), lambda i: (i, 0)),
        pl.BlockSpec((1, 1, BN), lambda i: (i, 0, 0)),
        w_spec((NUM_SPECIES, F)), w_spec((NUM_SPECIES, F)),
        w_spec((F, F)), w_spec((F, 1)),
    ]
    out_specs = [
        pl.BlockSpec((BN, F), lambda i: (i, 0)),
        pl.BlockSpec((BN, 1), lambda i: (i, 0)),
    ]
    out_shape = [jax.ShapeDtypeStruct((N, F), F32),
                 jax.ShapeDtypeStruct((N, 1), F32)]
    args = [agg, nf, spec3d, wp, ws, wmix, wread]
    if has_next:
        in_specs.append(w_spec((F, F)))
        out_specs.append(pl.BlockSpec((2, BN, H), lambda i: (0, i, 0)))
        out_shape.append(jax.ShapeDtypeStruct((2, N, H), F32))
        args.append(wlin_next)
    return pl.pallas_call(
        functools.partial(_node_body, has_next),
        grid=(grid,),
        in_specs=in_specs,
        out_specs=out_specs,
        out_shape=out_shape,
    )(*args)


def kernel(vectors, node_specie, senders, receivers, embed_table, W_lin,
           Wr1, Wr2, W_sh, W_prod_sp, W_skip_sp, W_mix, W_readout):
    spec3d = node_specie.astype(jnp.int32).reshape(N // BN, 1, BN)
    snd2d = senders.astype(jnp.int32).reshape(ROWS, CH)
    rcv2d = receivers.astype(jnp.int32).reshape(ROWS, CH)

    rg = _edge_factors(vectors, Wr1, Wr2, W_sh)
    embw0 = jnp.dot(embed_table, W_lin[0])
    nf, h2 = _embed(spec3d, embed_table, embw0)

    outs = []
    for l in range(NUM_LAYERS):
        agg = _sc_message(h2, rg[l], snd2d, rcv2d)
        wlin_next = W_lin[l + 1] if l + 1 < NUM_LAYERS else None
        res = _node_update(agg, nf, spec3d, W_prod_sp[l], W_skip_sp[l],
                           W_mix[l], W_readout[l], wlin_next)
        if wlin_next is not None:
            nf, out_l, h2 = res
        else:
            nf, out_l = res
        outs.append(out_l)
    return jnp.concatenate(outs, axis=1)


# speedup vs baseline: 2.0243x; 1.1978x over previous
import functools

import jax
import jax.numpy as jnp
from jax import lax
from jax.experimental import pallas as pl
from jax.experimental.pallas import tpu as pltpu
from jax.experimental.pallas import tpu_sc as plsc

N = 10000
E = 320000
F = 128
H = 64
NUM_SPECIES = 10
NUM_BESSEL = 8
R_MAX = 5.0
NUM_LAYERS = 2
AVG_NEIGH = 32.0
H_RAD = 64

NCORE = 2
NSUB = 16
CH = 125
ROWS = E // CH
CPS = ROWS // NSUB
IB = 40
NB = CPS // IB
RPS = N // NSUB

BE = 3200
BN = 2000
F32 = jnp.float32


def _edge_factor_body(vec_ref, wr1a_ref, wr2a_ref, wsha_ref,
                      wr1b_ref, wr2b_ref, wshb_ref, rg0_ref, rg1_ref):
    v = vec_ref[...]
    r2 = jnp.sum(v * v, axis=1, keepdims=True)
    r = jnp.sqrt(r2)
    rc = jnp.maximum(r, 1e-6)
    unit = v / rc
    sh = jnp.concatenate(
        [jnp.full((v.shape[0], 1), 0.28209479177387814, F32),
         0.4886025119029199 * unit], axis=1)
    n = (lax.broadcasted_iota(jnp.int32, (1, NUM_BESSEL), 1) + 1).astype(F32)
    b = jnp.sqrt(2.0 / R_MAX) * jnp.sin(n * (jnp.pi / R_MAX) * rc) / rc
    u = r / R_MAX
    u5 = u * u * u * u * u
    env = 1.0 - 21.0 * u5 + 35.0 * u5 * u - 15.0 * u5 * u * u
    env = jnp.where(u < 1.0, env, 0.0)
    rad = b * env

    for wr1_ref, wr2_ref, wsh_ref, out_ref in (
            (wr1a_ref, wr2a_ref, wsha_ref, rg0_ref),
            (wr1b_ref, wr2b_ref, wshb_ref, rg1_ref)):
        z = jnp.dot(rad, wr1_ref[...], preferred_element_type=F32)
        z = z * jax.nn.sigmoid(z)
        rw = jnp.dot(z, wr2_ref[...], preferred_element_type=F32)
        gate = jnp.dot(sh, wsh_ref[...], preferred_element_type=F32)
        out_ref[...] = rw * gate


def _edge_factors(vectors, Wr1, Wr2, W_sh):
    w_spec = lambda s: pl.BlockSpec(s, lambda i: (0,) * len(s))
    grid = E // BE
    return pl.pallas_call(
        _edge_factor_body,
        grid=(grid,),
        in_specs=[
            pl.BlockSpec((BE, 3), lambda i: (i, 0)),
            w_spec((NUM_BESSEL, H_RAD)), w_spec((H_RAD, F)), w_spec((4, F)),
            w_spec((NUM_BESSEL, H_RAD)), w_spec((H_RAD, F)), w_spec((4, F)),
        ],
        out_specs=[
            pl.BlockSpec((BE, F), lambda i: (i, 0)),
            pl.BlockSpec((BE, F), lambda i: (i, 0)),
        ],
        out_shape=[jax.ShapeDtypeStruct((E, F), F32)] * 2,
    )(vectors, Wr1[0], Wr2[0], W_sh[0], Wr1[1], Wr2[1], W_sh[1])


def _embed_body(spec_ref, emb_ref, embw_ref, nf_ref, h_ref):
    s = spec_ref[0, 0, :]
    oh = (s[:, None] == lax.broadcasted_iota(
        jnp.int32, (s.shape[0], NUM_SPECIES), 1)).astype(F32)
    nf_ref[...] = jnp.dot(oh, emb_ref[...], preferred_element_type=F32)
    h_ref[...] = jnp.dot(oh, embw_ref[...], preferred_element_type=F32)


def _embed(spec3d, embed_table, embw0):
    grid = N // BN
    return pl.pallas_call(
        _embed_body,
        grid=(grid,),
        in_specs=[
            pl.BlockSpec((1, 1, BN), lambda i: (i, 0, 0)),
            pl.BlockSpec((NUM_SPECIES, F), lambda i: (0, 0)),
            pl.BlockSpec((NUM_SPECIES, F), lambda i: (0, 0)),
        ],
        out_specs=[
            pl.BlockSpec((BN, F), lambda i: (i, 0)),
            pl.BlockSpec((BN, F), lambda i: (i, 0)),
        ],
        out_shape=[jax.ShapeDtypeStruct((N, F), F32),
                   jax.ShapeDtypeStruct((N, F), F32)],
    )(spec3d, embed_table, embw0)


def _sc_body(h_hbm, rg_hbm, snd_hbm, rcv_hbm, agg_hbm,
             h_sh, agg_sh, snd_v, rcv_v, rows_v, rg_v):
    cid = lax.axis_index("c")
    sid = lax.axis_index("s")

    pltpu.sync_copy(h_hbm.at[pl.ds(sid * RPS, RPS), pl.ds(cid * H, H)],
                    h_sh.at[pl.ds(sid * RPS, RPS)])

    def _zero_row(r, carry):
        for k in range(H // 16):
            rows_v[r, pl.ds(k * 16, 16)] = jnp.zeros((16,), F32)
        return carry
    lax.fori_loop(0, CH, _zero_row, 0)
    for t in range(RPS // CH):
        pltpu.sync_copy(rows_v, agg_sh.at[pl.ds(sid * RPS + t * CH, CH)])

    plsc.subcore_barrier()

    def _batch(b, carry):
        pltpu.sync_copy(snd_hbm.at[pl.ds(sid * CPS + b * IB, IB)], snd_v)
        pltpu.sync_copy(rcv_hbm.at[pl.ds(sid * CPS + b * IB, IB)], rcv_v)

        def _edge_chunk(j, c1):
            pltpu.sync_copy(h_sh.at[snd_v.at[j]], rows_v)
            pltpu.sync_copy(
                rg_hbm.at[pl.ds((sid * CPS + b * IB + j) * CH, CH),
                          pl.ds(cid * H, H)],
                rg_v)

            def _mul_row(r, c2):
                for k in range(H // 16):
                    sl = pl.ds(k * 16, 16)
                    rows_v[r, sl] = rows_v[r, sl] * rg_v[r, sl]
                return c2
            lax.fori_loop(0, CH, _mul_row, 0)

            pltpu.sync_copy(rows_v, agg_sh.at[rcv_v.at[j]], add=True)
            return c1
        lax.fori_loop(0, IB, _edge_chunk, 0)
        return carry
    lax.fori_loop(0, NB, _batch, 0)

    plsc.subcore_barrier()
    pltpu.sync_copy(agg_sh.at[pl.ds(sid * RPS, RPS)],
                    agg_hbm.at[pl.ds(sid * RPS, RPS), pl.ds(cid * H, H)])


def _sc_message(h2, rg, snd2d, rcv2d):
    mesh = plsc.VectorSubcoreMesh(core_axis_name="c", subcore_axis_name="s",
                                  num_cores=NCORE, num_subcores=NSUB)
    return pl.kernel(
        _sc_body,
        out_type=jax.ShapeDtypeStruct((N, F), F32),
        mesh=mesh,
        compiler_params=pltpu.CompilerParams(use_tc_tiling_on_sc=False),
        scratch_types=[
            pltpu.VMEM_SHARED((N, H), F32),
            pltpu.VMEM_SHARED((N, H), F32),
            pltpu.VMEM((IB, CH), jnp.int32),
            pltpu.VMEM((IB, CH), jnp.int32),
            pltpu.VMEM((CH, H), F32),
            pltpu.VMEM((CH, H), F32),
        ],
    )(h2, rg, snd2d, rcv2d)


def _node_body(has_next, agg_ref, nf_ref, spec_ref, wp_ref, ws_ref,
               wmix_ref, wread_ref, *rest):
    if has_next:
        wlin_ref, nfo_ref, out_ref, hn_ref = rest
    else:
        nfo_ref, out_ref = rest
    a = agg_ref[...] * (1.0 / AVG_NEIGH)
    poly = a + a * a + a * a * a
    s = spec_ref[0, 0, :]
    oh = (s[:, None] == lax.broadcasted_iota(
        jnp.int32, (s.shape[0], NUM_SPECIES), 1)).astype(F32)
    wp = jnp.dot(oh, wp_ref[...], preferred_element_type=F32)
    ws = jnp.dot(oh, ws_ref[...], preferred_element_type=F32)
    node_new = jnp.dot(poly * wp, wmix_ref[...], preferred_element_type=F32)
    nf = node_new + nf_ref[...] * ws
    nfo_ref[...] = nf
    out_ref[...] = jnp.dot(nf, wread_ref[...], preferred_element_type=F32)
    if has_next:
        hn_ref[...] = jnp.dot(nf, wlin_ref[...], preferred_element_type=F32)


def _node_update(agg, nf, spec3d, wp, ws, wmix, wread, wlin_next):
    has_next = wlin_next is not None
    grid = N // BN
    w_spec = lambda s: pl.BlockSpec(s, lambda i: (0,) * len(s))
    in_specs = [
        pl.BlockSpec((BN, F), lambda i: (i, 0)),
        pl.BlockSpec((BN, F), lambda i: (i, 0)),
        pl.BlockSpec((1, 1, BN), lambda i: (i, 0, 0)),
        w_spec((NUM_SPECIES, F)), w_spec((NUM_SPECIES, F)),
        w_spec((F, F)), w_spec((F, 1)),
    ]
    out_specs = [
        pl.BlockSpec((BN, F), lambda i: (i, 0)),
        pl.BlockSpec((BN, 1), lambda i: (i, 0)),
    ]
    out_shape = [jax.ShapeDtypeStruct((N, F), F32),
                 jax.ShapeDtypeStruct((N, 1), F32)]
    args = [agg, nf, spec3d, wp, ws, wmix, wread]
    if has_next:
        in_specs.append(w_spec((F, F)))
        out_specs.append(pl.BlockSpec((BN, F), lambda i: (i, 0)))
        out_shape.append(jax.ShapeDtypeStruct((N, F), F32))
        args.append(wlin_next)
    return pl.pallas_call(
        functools.partial(_node_body, has_next),
        grid=(grid,),
        in_specs=in_specs,
        out_specs=out_specs,
        out_shape=out_shape,
    )(*args)


def kernel(vectors, node_specie, senders, receivers, embed_table, W_lin,
           Wr1, Wr2, W_sh, W_prod_sp, W_skip_sp, W_mix, W_readout):
    spec3d = node_specie.astype(jnp.int32).reshape(N // BN, 1, BN)
    snd2d = senders.astype(jnp.int32).reshape(ROWS, CH)
    rcv2d = receivers.astype(jnp.int32).reshape(ROWS, CH)

    rg = _edge_factors(vectors, Wr1, Wr2, W_sh)
    embw0 = jnp.dot(embed_table, W_lin[0])
    nf, h2 = _embed(spec3d, embed_table, embw0)

    outs = []
    for l in range(NUM_LAYERS):
        agg = _sc_message(h2, rg[l], snd2d, rcv2d)
        wlin_next = W_lin[l + 1] if l + 1 < NUM_LAYERS else None
        res = _node_update(agg, nf, spec3d, W_prod_sp[l], W_skip_sp[l],
                           W_mix[l], W_readout[l], wlin_next)
        if wlin_next is not None:
            nf, out_l, h2 = res
        else:
            nf, out_l = res
        outs.append(out_l)
    return jnp.concatenate(outs, axis=1)


# speedup vs baseline: 7.3347x; 3.6233x over previous
import functools

import jax
import jax.numpy as jnp
from jax import lax
from jax.experimental import pallas as pl
from jax.experimental.pallas import tpu as pltpu
from jax.experimental.pallas import tpu_sc as plsc

N = 10000
E = 320000
F = 128
H = 64
NUM_SPECIES = 10
NUM_BESSEL = 8
R_MAX = 5.0
NUM_LAYERS = 2
AVG_NEIGH = 32.0
H_RAD = 64

NCORE = 2
NSUB = 16
CH = 125
ROWS = E // CH
CPS = ROWS // NSUB
IB = 20
NB = CPS // IB
RPS = N // NSUB

BE = 6400
BN = 2000
F32 = jnp.float32


def _edge_factor_body(vt_ref, wr1_ref, wr2_ref, wsh_ref, out_ref):
    vx = vt_ref[0:1, :]
    vy = vt_ref[1:2, :]
    vz = vt_ref[2:3, :]
    r2 = vx * vx + vy * vy + vz * vz
    r = jnp.sqrt(r2)
    rc = jnp.maximum(r, 1e-6)
    inv = 1.0 / rc
    u = r * (1.0 / R_MAX)
    u2 = u * u
    u5 = u2 * u2 * u
    env = 1.0 + u5 * (-21.0 + u * (35.0 - 15.0 * u))
    env = jnp.where(u < 1.0, env, 0.0)
    scale = jnp.sqrt(2.0 / R_MAX) * inv * env

    x = (jnp.pi / R_MAX) * rc
    s1 = jnp.sin(x)
    two_c = 2.0 * jnp.cos(x)
    sines = [s1]
    s_prev, s_cur = jnp.zeros_like(s1), s1
    for _ in range(NUM_BESSEL - 1):
        s_prev, s_cur = s_cur, two_c * s_cur - s_prev
        sines.append(s_cur)
    radT = jnp.concatenate([scale * s for s in sines], axis=0)

    c0 = jnp.full(vx.shape, 0.28209479177387814, F32)
    c1 = 0.4886025119029199 * inv
    shT = jnp.concatenate([c0, c1 * vx, c1 * vy, c1 * vz], axis=0)

    dn = (((0,), (0,)), ((), ()))
    z = lax.dot_general(radT, wr1_ref[...], dn, preferred_element_type=F32)
    z = z * jax.nn.sigmoid(z)
    rw = jnp.dot(z, wr2_ref[...], preferred_element_type=F32)
    gate = lax.dot_general(shT, wsh_ref[...], dn, preferred_element_type=F32)
    out_ref[...] = rw * gate


def _edge_factors(vt, wr1, wr2, wsh):
    w_spec = lambda s: pl.BlockSpec(s, lambda i: (0,) * len(s))
    return pl.pallas_call(
        _edge_factor_body,
        grid=(E // BE,),
        in_specs=[
            pl.BlockSpec((3, BE), lambda i: (0, i)),
            w_spec((NUM_BESSEL, H_RAD)), w_spec((H_RAD, F)), w_spec((4, F)),
        ],
        out_specs=pl.BlockSpec((BE, F), lambda i: (i, 0)),
        out_shape=jax.ShapeDtypeStruct((E, F), F32),
    )(vt, wr1, wr2, wsh)


def _embed_body(spec_ref, emb_ref, embw_ref, nf_ref, h_ref):
    s = spec_ref[0, 0, :]
    oh = (s[:, None] == lax.broadcasted_iota(
        jnp.int32, (s.shape[0], NUM_SPECIES), 1)).astype(F32)
    nf_ref[...] = jnp.dot(oh, emb_ref[...], preferred_element_type=F32)
    h_ref[...] = jnp.dot(oh, embw_ref[...], preferred_element_type=F32)


def _embed(spec3d, embed_table, embw0):
    return pl.pallas_call(
        _embed_body,
        grid=(N // BN,),
        in_specs=[
            pl.BlockSpec((1, 1, BN), lambda i: (i, 0, 0)),
            pl.BlockSpec((NUM_SPECIES, F), lambda i: (0, 0)),
            pl.BlockSpec((NUM_SPECIES, F), lambda i: (0, 0)),
        ],
        out_specs=[
            pl.BlockSpec((BN, F), lambda i: (i, 0)),
            pl.BlockSpec((BN, F), lambda i: (i, 0)),
        ],
        out_shape=[jax.ShapeDtypeStruct((N, F), F32),
                   jax.ShapeDtypeStruct((N, F), F32)],
    )(spec3d, embed_table, embw0)


def _sc_body(h_hbm, rg_hbm, snd_hbm, rcv_hbm, agg_hbm,
             h_sh, agg_sh, snd_v, rcv_v,
             rows_a, rows_b, rg_a, rg_b, sems):
    cid = lax.axis_index("c")
    sid = lax.axis_index("s")

    pltpu.sync_copy(h_hbm.at[pl.ds(sid * RPS, RPS), pl.ds(cid * H, H)],
                    h_sh.at[pl.ds(sid * RPS, RPS)])

    def _zero_row(r, carry):
        for k in range(H // 16):
            rows_a[r, pl.ds(k * 16, 16)] = jnp.zeros((16,), F32)
        return carry
    lax.fori_loop(0, CH, _zero_row, 0)
    for t in range(RPS // CH):
        pltpu.sync_copy(rows_a, agg_sh.at[pl.ds(sid * RPS + t * CH, CH)])

    plsc.subcore_barrier()

    rows_bufs = (rows_a, rows_b)
    rg_bufs = (rg_a, rg_b)

    def _issue(j, t):
        bi = t % 2
        g = pltpu.async_copy(h_sh.at[snd_v.at[t % IB]], rows_bufs[bi],
                             sems.at[bi])
        l = pltpu.async_copy(
            rg_hbm.at[pl.ds(j * CH, CH), pl.ds(cid * H, H)],
            rg_bufs[bi], sems.at[2 + bi])
        return g, l

    def _mul(bi):
        rows_v, rg_v = rows_bufs[bi], rg_bufs[bi]

        @functools.partial(plsc.parallel_loop, 0, CH, unroll=5)
        def _mul_row(r):
            for k in range(H // 16):
                sl = pl.ds(k * 16, 16)
                rows_v[r, sl] = rows_v[r, sl] * rg_v[r, sl]

    def _batch(b, carry):
        base = sid * CPS + b * IB
        pltpu.sync_copy(snd_hbm.at[pl.ds(base, IB)], snd_v)
        pltpu.sync_copy(rcv_hbm.at[pl.ds(base, IB)], rcv_v)

        pend = [None, None]
        descs = _issue(base, 0)
        for t in range(IB):
            bi = t % 2
            if t + 1 < IB:
                nxt = _issue(base + t + 1, t + 1)
            descs[0].wait()
            descs[1].wait()
            _mul(bi)
            pend[bi] = pltpu.async_copy(
                rows_bufs[bi], agg_sh.at[rcv_v.at[t]], sems.at[4 + bi],
                add=True)
            if t + 1 < IB:
                if pend[1 - bi] is not None:
                    pend[1 - bi].wait()
                    pend[1 - bi] = None
                descs = nxt
        for p in pend:
            if p is not None:
                p.wait()
        return carry
    lax.fori_loop(0, NB, _batch, 0)

    plsc.subcore_barrier()
    pltpu.sync_copy(agg_sh.at[pl.ds(sid * RPS, RPS)],
                    agg_hbm.at[pl.ds(sid * RPS, RPS), pl.ds(cid * H, H)])


def _sc_message(h2, rg, snd2d, rcv2d):
    mesh = plsc.VectorSubcoreMesh(core_axis_name="c", subcore_axis_name="s",
                                  num_cores=NCORE, num_subcores=NSUB)
    return pl.kernel(
        _sc_body,
        out_type=jax.ShapeDtypeStruct((N, F), F32),
        mesh=mesh,
        compiler_params=pltpu.CompilerParams(use_tc_tiling_on_sc=False),
        scratch_types=[
            pltpu.VMEM_SHARED((N, H), F32),
            pltpu.VMEM_SHARED((N, H), F32),
            pltpu.VMEM((IB, CH), jnp.int32),
            pltpu.VMEM((IB, CH), jnp.int32),
            pltpu.VMEM((CH, H), F32),
            pltpu.VMEM((CH, H), F32),
            pltpu.VMEM((CH, H), F32),
            pltpu.VMEM((CH, H), F32),
            pltpu.SemaphoreType.DMA((6,)),
        ],
    )(h2, rg, snd2d, rcv2d)


def _node_body(has_next, agg_ref, nf_ref, spec_ref, wp_ref, ws_ref,
               wmix_ref, wread_ref, *rest):
    if has_next:
        wlin_ref, nfo_ref, out_ref, hn_ref = rest
    else:
        nfo_ref, out_ref = rest
    a = agg_ref[...] * (1.0 / AVG_NEIGH)
    poly = a + a * a + a * a * a
    s = spec_ref[0, 0, :]
    oh = (s[:, None] == lax.broadcasted_iota(
        jnp.int32, (s.shape[0], NUM_SPECIES), 1)).astype(F32)
    wp = jnp.dot(oh, wp_ref[...], preferred_element_type=F32)
    ws = jnp.dot(oh, ws_ref[...], preferred_element_type=F32)
    node_new = jnp.dot(poly * wp, wmix_ref[...], preferred_element_type=F32)
    nf = node_new + nf_ref[...] * ws
    nfo_ref[...] = nf
    out_ref[...] = jnp.dot(nf, wread_ref[...], preferred_element_type=F32)
    if has_next:
        hn_ref[...] = jnp.dot(nf, wlin_ref[...], preferred_element_type=F32)


def _node_update(agg, nf, spec3d, wp, ws, wmix, wread, wlin_next):
    has_next = wlin_next is not None
    w_spec = lambda s: pl.BlockSpec(s, lambda i: (0,) * len(s))
    in_specs = [
        pl.BlockSpec((BN, F), lambda i: (i, 0)),
        pl.BlockSpec((BN, F), lambda i: (i, 0)),
        pl.BlockSpec((1, 1, BN), lambda i: (i, 0, 0)),
        w_spec((NUM_SPECIES, F)), w_spec((NUM_SPECIES, F)),
        w_spec((F, F)), w_spec((F, 1)),
    ]
    out_specs = [
        pl.BlockSpec((BN, F), lambda i: (i, 0)),
        pl.BlockSpec((BN, 1), lambda i: (i, 0)),
    ]
    out_shape = [jax.ShapeDtypeStruct((N, F), F32),
                 jax.ShapeDtypeStruct((N, 1), F32)]
    args = [agg, nf, spec3d, wp, ws, wmix, wread]
    if has_next:
        in_specs.append(w_spec((F, F)))
        out_specs.append(pl.BlockSpec((BN, F), lambda i: (i, 0)))
        out_shape.append(jax.ShapeDtypeStruct((N, F), F32))
        args.append(wlin_next)
    return pl.pallas_call(
        functools.partial(_node_body, has_next),
        grid=(N // BN,),
        in_specs=in_specs,
        out_specs=out_specs,
        out_shape=out_shape,
    )(*args)


def kernel(vectors, node_specie, senders, receivers, embed_table, W_lin,
           Wr1, Wr2, W_sh, W_prod_sp, W_skip_sp, W_mix, W_readout):
    spec3d = node_specie.astype(jnp.int32).reshape(N // BN, 1, BN)
    snd2d = senders.astype(jnp.int32).reshape(ROWS, CH)
    rcv2d = receivers.astype(jnp.int32).reshape(ROWS, CH)
    vt = vectors.T

    rg = [_edge_factors(vt, Wr1[l], Wr2[l], W_sh[l])
          for l in range(NUM_LAYERS)]
    embw0 = jnp.dot(embed_table, W_lin[0])
    nf, h2 = _embed(spec3d, embed_table, embw0)

    outs = []
    for l in range(NUM_LAYERS):
        agg = _sc_message(h2, rg[l], snd2d, rcv2d)
        wlin_next = W_lin[l + 1] if l + 1 < NUM_LAYERS else None
        res = _node_update(agg, nf, spec3d, W_prod_sp[l], W_skip_sp[l],
                           W_mix[l], W_readout[l], wlin_next)
        if wlin_next is not None:
            nf, out_l, h2 = res
        else:
            nf, out_l = res
        outs.append(out_l)
    return jnp.concatenate(outs, axis=1)
